# baseline (speedup 1.0000x reference)
import functools

import jax
import jax.numpy as jnp
from jax.experimental import pallas as pl
from jax.experimental.pallas import tpu as pltpu

_NB = 1024
_DEF = jax.lax.Precision.DEFAULT
_INTERPRET = False


def _monotonic_key(f):
    ib = jax.lax.bitcast_convert_type(f, jnp.int32)
    return jnp.where(ib >= 0, ib, jnp.int32(-2147483648) - ib)


def _body(K, NB, NJ, q_ref, wk_ref, xs_ref, xa_ref, tok_ref, mask_ref, key_ref):
    r = pl.program_id(0)
    j = pl.program_id(1)
    i32 = jnp.int32
    N = NJ * NB
    p = jax.lax.rem(r, 2)
    wbase = pl.multiple_of(p * N + j * NB, NB)
    rbase = pl.multiple_of((1 - p) * N, N)

    k_blk = jax.lax.dot_general(xs_ref[0], wk_ref[...], (((1,), (1,)), ((), ())),
                                precision=_DEF,
                                preferred_element_type=jnp.float32)
    s_row = jax.lax.dot_general(q_ref[...], k_blk, (((1,), (1,)), ((), ())),
                                precision=_DEF,
                                preferred_element_type=jnp.float32)
    key_ref[0:1, pl.ds(wbase, NB)] = _monotonic_key(s_row)

    key = key_ref[0:1, pl.ds(rbase, N)]
    kK = i32(K)

    def bis(_, c):
        lo, hi = c
        mid = (lo >> 1) + (hi >> 1) + ((lo | hi) & 1)
        cnt = jnp.sum((key >= mid).astype(i32), axis=1, keepdims=True)
        ge = cnt >= kK
        return jnp.where(ge, mid, lo), jnp.where(ge, hi, mid - 1)

    lo0 = jnp.full((1, 1), -2147483648, i32)
    hi0 = jnp.full((1, 1), 2147483647, i32)
    t, _ = jax.lax.fori_loop(0, 33, bis, (lo0, hi0))

    gt = key > t
    need = kK - jnp.sum(gt.astype(i32), axis=1, keepdims=True)
    eq = key == t
    idx = jax.lax.broadcasted_iota(i32, (1, N), 1)

    def bis2(_, c):
        lo, hi = c
        mid = (lo + hi) >> 1
        cnt = jnp.sum((eq & (idx < mid)).astype(i32), axis=1, keepdims=True)
        ge = cnt >= need
        return jnp.where(ge, lo, mid + 1), jnp.where(ge, mid, hi)

    cc, _ = jax.lax.fori_loop(
        0, 13, bis2, (jnp.zeros((1, 1), i32), jnp.full((1, 1), N, i32)))

    mask_ref[0] = (gt | (eq & (idx < cc))).astype(jnp.float32)

    keyr = key_ref[0:1, pl.ds(pl.multiple_of(rbase + j * NB, NB), NB)]
    idxr = jax.lax.broadcasted_iota(i32, (1, NB), 1) + j * NB
    m_row = (keyr > t) | ((keyr == t) & (idxr < cc))
    m_col = jnp.transpose(m_row.astype(jnp.float32))
    tok_ref[0] = xa_ref[0] * m_col


def kernel(x, learned_query, Wq, bq, Wk, bk):
    B, N, D = x.shape
    K = max(1, int(N * 0.5))
    NB = _NB
    NJ = N // NB
    q = jnp.dot(learned_query[0], Wq.T, precision=_DEF) + bq

    tok, mask = pl.pallas_call(
        functools.partial(_body, K, NB, NJ),
        grid=(B + 1, NJ),
        in_specs=[
            pl.BlockSpec((1, D), lambda r, j: (0, 0)),
            pl.BlockSpec((D, D), lambda r, j: (0, 0)),
            pl.BlockSpec((1, NB, D), lambda r, j: (jnp.minimum(r, B - 1), j, 0)),
            pl.BlockSpec((1, NB, D), lambda r, j: (jnp.maximum(r - 1, 0), j, 0)),
        ],
        out_specs=[
            pl.BlockSpec((1, NB, D), lambda r, j: (jnp.maximum(r - 1, 0), j, 0)),
            pl.BlockSpec((1, 1, N), lambda r, j: (jnp.maximum(r - 1, 0), 0, 0)),
        ],
        out_shape=[
            jax.ShapeDtypeStruct((B, N, D), jnp.float32),
            jax.ShapeDtypeStruct((B, 1, N), jnp.float32),
        ],
        scratch_shapes=[pltpu.VMEM((1, 2 * N), jnp.int32)],
        interpret=_INTERPRET,
    )(q, Wk, x, x)
    return tok, mask.reshape(B, N)


# speedup vs baseline: 1.5292x; 1.5292x over previous
import functools

import jax
import jax.numpy as jnp
from jax.experimental import pallas as pl
from jax.experimental.pallas import tpu as pltpu

_NB = 1024
_DEF = jax.lax.Precision.DEFAULT
_INTERPRET = False


def _monotonic_key(f):
    ib = jax.lax.bitcast_convert_type(f, jnp.int32)
    return jnp.where(ib >= 0, ib, jnp.int32(-2147483648) - ib)


def _body(K, NB, NJ, q_ref, wk_ref, xs_ref, xa_ref, tok_ref, mask_ref, key_ref,
          tc_ref):
    r = pl.program_id(0)
    j = pl.program_id(1)
    i32 = jnp.int32
    N = NJ * NB
    p = jax.lax.rem(r, 2)
    wbase = pl.multiple_of(p * N + j * NB, NB)
    rbase = pl.multiple_of((1 - p) * N, N)

    k_blk = jax.lax.dot_general(xs_ref[0], wk_ref[...], (((1,), (1,)), ((), ())),
                                precision=_DEF,
                                preferred_element_type=jnp.float32)
    s_row = jax.lax.dot_general(q_ref[...], k_blk, (((1,), (1,)), ((), ())),
                                precision=_DEF,
                                preferred_element_type=jnp.float32)
    key_ref[0:1, pl.ds(wbase, NB)] = _monotonic_key(s_row)

    @pl.when(j == 0)
    def _select():
        key = key_ref[0:1, pl.ds(rbase, N)]
        kK = i32(K)

        def bis(_, c):
            lo, hi = c
            mid = (lo >> 1) + (hi >> 1) + ((lo | hi) & 1)
            cnt = jnp.sum((key >= mid).astype(i32), axis=1, keepdims=True)
            ge = cnt >= kK
            return jnp.where(ge, mid, lo), jnp.where(ge, hi, mid - 1)

        lo0 = jnp.full((1, 1), -2147483648, i32)
        hi0 = jnp.full((1, 1), 2147483647, i32)
        t, _ = jax.lax.fori_loop(0, 33, bis, (lo0, hi0))

        gt = key > t
        need = kK - jnp.sum(gt.astype(i32), axis=1, keepdims=True)
        eq = key == t
        idx = jax.lax.broadcasted_iota(i32, (1, N), 1)

        def bis2(_, c):
            lo, hi = c
            mid = (lo + hi) >> 1
            cnt = jnp.sum((eq & (idx < mid)).astype(i32), axis=1,
                          keepdims=True)
            ge = cnt >= need
            return jnp.where(ge, lo, mid + 1), jnp.where(ge, mid, hi)

        cc, _ = jax.lax.fori_loop(
            0, 13, bis2, (jnp.zeros((1, 1), i32), jnp.full((1, 1), N, i32)))

        mask_ref[0] = (gt | (eq & (idx < cc))).astype(jnp.float32)
        tc_ref[0:1, :] = jnp.broadcast_to(t, (1, 128))
        tc_ref[1:2, :] = jnp.broadcast_to(cc, (1, 128))

    t = tc_ref[0:1, 0:1]
    cc = tc_ref[1:2, 0:1]
    keyr = key_ref[0:1, pl.ds(pl.multiple_of(rbase + j * NB, NB), NB)]
    idxr = jax.lax.broadcasted_iota(i32, (1, NB), 1) + j * NB
    m_row = (keyr > t) | ((keyr == t) & (idxr < cc))
    m_col = jnp.transpose(m_row.astype(jnp.float32))
    tok_ref[0] = xa_ref[0] * m_col


def kernel(x, learned_query, Wq, bq, Wk, bk):
    B, N, D = x.shape
    K = max(1, int(N * 0.5))
    NB = _NB
    NJ = N // NB
    q = jnp.dot(learned_query[0], Wq.T, precision=_DEF) + bq

    tok, mask = pl.pallas_call(
        functools.partial(_body, K, NB, NJ),
        grid=(B + 1, NJ),
        in_specs=[
            pl.BlockSpec((1, D), lambda r, j: (0, 0)),
            pl.BlockSpec((D, D), lambda r, j: (0, 0)),
            pl.BlockSpec((1, NB, D), lambda r, j: (jnp.minimum(r, B - 1), j, 0)),
            pl.BlockSpec((1, NB, D), lambda r, j: (jnp.maximum(r - 1, 0), j, 0)),
        ],
        out_specs=[
            pl.BlockSpec((1, NB, D), lambda r, j: (jnp.maximum(r - 1, 0), j, 0)),
            pl.BlockSpec((1, 1, N), lambda r, j: (jnp.maximum(r - 1, 0), 0, 0)),
        ],
        out_shape=[
            jax.ShapeDtypeStruct((B, N, D), jnp.float32),
            jax.ShapeDtypeStruct((B, 1, N), jnp.float32),
        ],
        scratch_shapes=[pltpu.VMEM((1, 2 * N), jnp.int32),
                        pltpu.VMEM((2, 128), jnp.int32)],
        interpret=_INTERPRET,
    )(q, Wk, x, x)
    return tok, mask.reshape(B, N)


# speedup vs baseline: 1.7710x; 1.1582x over previous
import functools

import jax
import jax.numpy as jnp
from jax.experimental import pallas as pl
from jax.experimental.pallas import tpu as pltpu

_NB = 2048
_DEF = jax.lax.Precision.DEFAULT
_INTERPRET = False


def _monotonic_key(f):
    ib = jax.lax.bitcast_convert_type(f, jnp.int32)
    return jnp.where(ib >= 0, ib, jnp.int32(-2147483648) - ib)


def _scores_body(x_ref, q_ref, wk_ref, st_ref):
    k_blk = jax.lax.dot_general(x_ref[0], wk_ref[...], (((1,), (1,)), ((), ())),
                                precision=_DEF,
                                preferred_element_type=jnp.float32)
    s_row = jax.lax.dot_general(q_ref[...], k_blk, (((1,), (1,)), ((), ())),
                                precision=_DEF,
                                preferred_element_type=jnp.float32)
    st_ref[0, 0] = s_row


def _select_apply_body(K, NB, s_ref, st_ref, x_ref, tok_ref, mask_ref, tc_ref):
    b = pl.program_id(0)
    j = pl.program_id(1)
    B, N = s_ref.shape
    i32 = jnp.int32

    @pl.when((b == 0) & (j == 0))
    def _select():
        key = _monotonic_key(s_ref[...])
        kK = i32(K)

        def bis(_, c):
            lo, hi = c
            mid = (lo >> 1) + (hi >> 1) + ((lo | hi) & 1)
            cnt = jnp.sum((key >= mid).astype(i32), axis=1, keepdims=True)
            ge = cnt >= kK
            return jnp.where(ge, mid, lo), jnp.where(ge, hi, mid - 1)

        lo0 = jnp.full((B, 1), -2147483648, i32)
        hi0 = jnp.full((B, 1), 2147483647, i32)
        t, _ = jax.lax.fori_loop(0, 33, bis, (lo0, hi0))

        gt = key > t
        need = kK - jnp.sum(gt.astype(i32), axis=1, keepdims=True)
        eq = key == t
        idx = jax.lax.broadcasted_iota(i32, (B, N), 1)

        def bis2(_, c):
            lo, hi = c
            mid = (lo + hi) >> 1
            cnt = jnp.sum((eq & (idx < mid)).astype(i32), axis=1, keepdims=True)
            ge = cnt >= need
            return jnp.where(ge, lo, mid + 1), jnp.where(ge, mid, hi)

        cc, _ = jax.lax.fori_loop(
            0, 13, bis2, (jnp.zeros((B, 1), i32), jnp.full((B, 1), N, i32)))

        mask_ref[...] = (gt | (eq & (idx < cc))).astype(jnp.float32)
        eye = (jax.lax.broadcasted_iota(i32, (B, B), 0) ==
               jax.lax.broadcasted_iota(i32, (B, B), 1))
        tc_ref[0:1, :] = jnp.sum(
            jnp.where(eye, jnp.broadcast_to(t, (B, B)), 0), axis=0, keepdims=True)
        tc_ref[1:2, :] = jnp.sum(
            jnp.where(eye, jnp.broadcast_to(cc, (B, B)), 0), axis=0, keepdims=True)

    onehot = jax.lax.broadcasted_iota(i32, (1, B), 1) == b
    t_b = jnp.sum(jnp.where(onehot, tc_ref[0:1, :], 0), axis=1, keepdims=True)
    c_b = jnp.sum(jnp.where(onehot, tc_ref[1:2, :], 0), axis=1, keepdims=True)
    keyr = _monotonic_key(st_ref[0, 0])
    idxr = jax.lax.broadcasted_iota(i32, (1, NB), 1) + j * NB
    m_row = (keyr > t_b) | ((keyr == t_b) & (idxr < c_b))
    m_col = jnp.transpose(m_row.astype(jnp.float32))
    tok_ref[0] = x_ref[0] * m_col


def kernel(x, learned_query, Wq, bq, Wk, bk):
    B, N, D = x.shape
    K = max(1, int(N * 0.5))
    NB = _NB
    nj = N // NB
    q = jnp.dot(learned_query[0], Wq.T, precision=_DEF) + bq

    st = pl.pallas_call(
        _scores_body,
        grid=(B, nj),
        in_specs=[
            pl.BlockSpec((1, NB, D), lambda b, j: (b, j, 0)),
            pl.BlockSpec((1, D), lambda b, j: (0, 0)),
            pl.BlockSpec((D, D), lambda b, j: (0, 0)),
        ],
        out_specs=pl.BlockSpec((1, 1, 1, NB), lambda b, j: (b, j, 0, 0)),
        out_shape=jax.ShapeDtypeStruct((B, nj, 1, NB), jnp.float32),
        compiler_params=pltpu.CompilerParams(
            dimension_semantics=("parallel", "parallel")),
        interpret=_INTERPRET,
    )(x, q, Wk)

    s = st.reshape(B, N)

    tok, mask = pl.pallas_call(
        functools.partial(_select_apply_body, K, NB),
        grid=(B, nj),
        in_specs=[
            pl.BlockSpec((B, N), lambda b, j: (0, 0)),
            pl.BlockSpec((1, 1, 1, NB), lambda b, j: (b, j, 0, 0)),
            pl.BlockSpec((1, NB, D), lambda b, j: (b, j, 0)),
        ],
        out_specs=[
            pl.BlockSpec((1, NB, D), lambda b, j: (b, j, 0)),
            pl.BlockSpec((B, N), lambda b, j: (0, 0)),
        ],
        out_shape=[
            jax.ShapeDtypeStruct((B, N, D), jnp.float32),
            jax.ShapeDtypeStruct((B, N), jnp.float32),
        ],
        scratch_shapes=[pltpu.VMEM((2, B), jnp.int32)],
        interpret=_INTERPRET,
    )(s, st, x)
    return tok, mask
